# Initial kernel scaffold; baseline (speedup 1.0000x reference)
import jax
import jax.numpy as jnp
from jax.experimental import pallas as pl


def kernel(x, edge_index, batch, params):
    raise NotImplementedError("write your pallas kernel here")


# speedup vs baseline: 12.0277x; 12.0277x over previous
import functools
import jax
import jax.numpy as jnp
from jax import lax
from jax.experimental import pallas as pl
from jax.experimental.pallas import tpu as pltpu
from jax.experimental.pallas import tpu_sc as plsc

N = 10000
H = 128
NUM_LAYERS = 3
RES_FREQ = 2

NW = 32
CH = 128
E2 = 640000
EP = 643072
PAD = EP - E2
EPW = EP // NW
NCH = EPW // CH
ZR = 10240
DUMMY = N
RPT = ZR // 16
ORB = 624
CW = 128

_SC_MESH = plsc.VectorSubcoreMesh(core_axis_name="c", subcore_axis_name="s")


def _zero_rows(ref, nrows, width):
    z = jnp.zeros((16,), jnp.float32)

    def body(i, _):
        for j in range(width // 16):
            ref[i, pl.ds(j * 16, 16)] = z
        return 0

    lax.fori_loop(0, nrows, body, 0)


def _staged_readout(acc, stage, out_hbm, c, s):
    for t in range(5):
        base = s * ORB + t * CH
        pltpu.sync_copy(acc.at[pl.ds(base, CH)], stage)
        pltpu.sync_copy(stage, out_hbm.at[c, pl.ds(base, CH)])


@functools.partial(
    pl.kernel,
    mesh=_SC_MESH,
    out_type=jax.ShapeDtypeStruct((2, N, H), jnp.float32),
    scratch_types=[
        pltpu.VMEM((CH,), jnp.int32),
        pltpu.VMEM((CH,), jnp.int32),
        pltpu.VMEM((CH, H), jnp.float32),
        pltpu.VMEM_SHARED((ZR, H), jnp.float32),
        pltpu.SemaphoreType.DMA,
    ],
)
def _sc_msg(y_hbm, src_hbm, dst_hbm, z_hbm, sidx, didx, rows, zsh, sem):
    c = lax.axis_index("c")
    s = lax.axis_index("s")
    wid = s * 2 + c

    _zero_rows(rows, CH, H)
    for t in range(RPT // CH):
        pltpu.sync_copy(rows, zsh.at[pl.ds(s * RPT + t * CH, CH)])
    plsc.subcore_barrier()

    def body(i, _):
        off = wid * EPW + i * CH
        pltpu.sync_copy(src_hbm.at[pl.ds(off, CH)], sidx)
        pltpu.sync_copy(dst_hbm.at[pl.ds(off, CH)], didx)
        pltpu.async_copy(y_hbm.at[sidx], rows, sem).wait()
        pltpu.sync_copy(rows, zsh.at[didx], add=True)
        return 0

    lax.fori_loop(0, NCH, body, 0)
    plsc.subcore_barrier()
    _staged_readout(zsh, rows, z_hbm, c, s)


@functools.partial(
    pl.kernel,
    mesh=_SC_MESH,
    out_type=jax.ShapeDtypeStruct((2, N, CW), jnp.float32),
    scratch_types=[
        pltpu.VMEM((CH,), jnp.int32),
        pltpu.VMEM((CH, CW), jnp.float32),
        pltpu.VMEM_SHARED((ZR, CW), jnp.float32),
    ],
)
def _sc_cnt(idx_hbm, c_hbm, cidx, obuf, csh):
    c = lax.axis_index("c")
    s = lax.axis_index("s")
    wid = s * 2 + c

    _zero_rows(obuf, CH, CW)
    for t in range(RPT // CH):
        pltpu.sync_copy(obuf, csh.at[pl.ds(s * RPT + t * CH, CH)])
    plsc.subcore_barrier()

    one = jnp.ones((16,), jnp.float32)

    def fill(i, _):
        for j in range(CW // 16):
            obuf[i, pl.ds(j * 16, 16)] = one
        return 0

    lax.fori_loop(0, CH, fill, 0)

    def body(i, _):
        off = wid * EPW + i * CH
        pltpu.sync_copy(idx_hbm.at[pl.ds(off, CH)], cidx)
        pltpu.sync_copy(obuf, csh.at[cidx], add=True)
        return 0

    lax.fori_loop(0, NCH, body, 0)
    plsc.subcore_barrier()
    _staged_readout(csh, obuf, c_hbm, c, s)


def _relu(v):
    return jnp.maximum(v, 0.0)


def _mm(a, b):
    return jnp.dot(a, b, preferred_element_type=jnp.float32)


def _tc_in_body(x, w1, b1, w2, b2, wg, cp, wr, oh, oy, od):
    h = _mm(_relu(_mm(x[...], w1[...]) + b1[...]), w2[...]) + b2[...]
    cnt = (cp[0] + cp[1])[:, 0:1]
    dinv = lax.rsqrt(wr[0, 0] * cnt + 2.0)
    oh[...] = h
    od[...] = dinv
    oy[...] = dinv * _mm(h, wg[...])


def _tc_in(x, w1, b1, w2, b2, wg, cp, wr):
    return pl.pallas_call(
        _tc_in_body,
        out_shape=(
            jax.ShapeDtypeStruct((N, H), jnp.float32),
            jax.ShapeDtypeStruct((N, H), jnp.float32),
            jax.ShapeDtypeStruct((N, 1), jnp.float32),
        ),
    )(x, w1, b1.reshape(1, H), w2, b2.reshape(1, H), wg, cp, wr)


def _tc_post_body(has_skip, is_final, *refs):
    if is_final:
        (zp, y, h, dinv, wr, gb, ms, nw, nb, skw, skb,
         hw1, hb1, hw2, hb2, hw3, hb3, out) = refs
    elif has_skip:
        (zp, y, h, dinv, wr, gb, ms, nw, nb, skw, skb, wgn, oh, oy) = refs
    else:
        (zp, y, h, dinv, wr, gb, ms, nw, nb, wgn, oh, oy) = refs
    di = dinv[...]
    g = di * (wr[0, 0] * (zp[0] + zp[1]) + 2.0 * y[...]) + gb[...]
    mean = jnp.mean(g, axis=0, keepdims=True)
    o = g - ms[...] * mean
    var = jnp.mean(o * o, axis=0, keepdims=True)
    g = nw[...] * o * lax.rsqrt(var + 1e-5) + nb[...]
    g = jnp.where(g > 0, g, jnp.exp(g) - 1.0)
    if has_skip:
        g = g + _mm(h[...], skw[...]) + skb[...]
    if is_final:
        q = _relu(_mm(g, hw1[...]) + hb1[...])
        q = _relu(_mm(q, hw2[...]) + hb2[...])
        q = _mm(q, hw3[...]) + hb3[...]
        out[...] = 1.0 / (1.0 + jnp.exp(-q))
    else:
        oh[...] = g
        oy[...] = di * _mm(g, wgn[...])


def _tc_mid(has_skip, args):
    return pl.pallas_call(
        functools.partial(_tc_post_body, has_skip, False),
        out_shape=(
            jax.ShapeDtypeStruct((N, H), jnp.float32),
            jax.ShapeDtypeStruct((N, H), jnp.float32),
        ),
    )(*args)


def _tc_final(args):
    return pl.pallas_call(
        functools.partial(_tc_post_body, True, True),
        out_shape=jax.ShapeDtypeStruct((N, 1), jnp.float32),
    )(*args)


def kernel(x, edge_index, batch, params):
    p = params
    wr = p['edge_weight'].reshape(1, 1)
    row, col = edge_index[0], edge_index[1]
    pad0 = jnp.zeros((PAD,), jnp.int32)
    padd = jnp.full((PAD,), DUMMY, jnp.int32)
    srcg = jnp.concatenate([row, col, pad0])
    cidx = jnp.concatenate([row, col, padd])
    dst = jnp.concatenate([col, row, padd])

    cp = _sc_cnt(cidx)
    h, y, dinv = _tc_in(x, p['in_W1'], p['in_b1'], p['in_W2'], p['in_b2'],
                        p['gcn_W0'], cp, wr)

    for i in range(NUM_LAYERS):
        zp = _sc_msg(y, srcg, dst)
        gb = p['gcn_b%d' % i].reshape(1, H)
        ms = p['norm_ms%d' % i].reshape(1, H)
        nw = p['norm_w%d' % i].reshape(1, H)
        nb = p['norm_b%d' % i].reshape(1, H)
        if i < NUM_LAYERS - 1:
            args = [zp, y, h, dinv, wr, gb, ms, nw, nb]
            has_skip = i % RES_FREQ == 0
            if has_skip:
                args += [p['skip_W%d' % i], p['skip_b%d' % i].reshape(1, H)]
            args += [p['gcn_W%d' % (i + 1)]]
            h, y = _tc_mid(has_skip, args)
        else:
            args = [zp, y, h, dinv, wr, gb, ms, nw, nb,
                    p['skip_W%d' % i], p['skip_b%d' % i].reshape(1, H),
                    p['head_W1'], p['head_b1'].reshape(1, H),
                    p['head_W2'], p['head_b2'].reshape(1, H // 2),
                    p['head_W3'], p['head_b3'].reshape(1, 1)]
            out = _tc_final(args)
    return out.reshape(-1)


# speedup vs baseline: 16.4184x; 1.3650x over previous
import functools
import jax
import jax.numpy as jnp
from jax import lax
from jax.experimental import pallas as pl
from jax.experimental.pallas import tpu as pltpu
from jax.experimental.pallas import tpu_sc as plsc

N = 10000
H = 128
NUM_LAYERS = 3
RES_FREQ = 2

NW = 32
CH = 128
E2 = 640000
EP = 643072
PAD = EP - E2
EPW = EP // NW
NCH = EPW // CH
ZR = 10240
DUMMY = N
RPT = ZR // 16
ORB = 624
CW = 128

_SC_MESH = plsc.VectorSubcoreMesh(core_axis_name="c", subcore_axis_name="s")


def _zero_rows(ref, nrows, width):
    z = jnp.zeros((16,), jnp.float32)

    def body(i, _):
        for j in range(width // 16):
            ref[i, pl.ds(j * 16, 16)] = z
        return 0

    lax.fori_loop(0, nrows, body, 0)


def _staged_readout(acc, stage, out_hbm, c, s):
    for t in range(5):
        base = s * ORB + t * CH
        pltpu.sync_copy(acc.at[pl.ds(base, CH)], stage)
        pltpu.sync_copy(stage, out_hbm.at[c, pl.ds(base, CH)])


@functools.partial(
    pl.kernel,
    mesh=_SC_MESH,
    out_type=jax.ShapeDtypeStruct((2, N, H), jnp.float32),
    scratch_types=[
        pltpu.VMEM((CH,), jnp.int32),
        pltpu.VMEM((CH,), jnp.int32),
        pltpu.VMEM((CH,), jnp.int32),
        pltpu.VMEM((CH,), jnp.int32),
        pltpu.VMEM((CH, H), jnp.float32),
        pltpu.VMEM((CH, H), jnp.float32),
        pltpu.VMEM_SHARED((ZR, H), jnp.float32),
        pltpu.SemaphoreType.DMA,
        pltpu.SemaphoreType.DMA,
    ],
)
def _sc_msg(y_hbm, src_hbm, dst_hbm, z_hbm, sidx0, didx0, sidx1, didx1,
            rows0, rows1, zsh, sem0, sem1):
    c = lax.axis_index("c")
    s = lax.axis_index("s")
    wid = s * 2 + c
    base = wid * EPW

    _zero_rows(rows0, CH, H)
    for t in range(RPT // CH):
        pltpu.sync_copy(rows0, zsh.at[pl.ds(s * RPT + t * CH, CH)])
    plsc.subcore_barrier()

    def load_idx(off, sidx, didx):
        pltpu.sync_copy(src_hbm.at[pl.ds(off, CH)], sidx)
        pltpu.sync_copy(dst_hbm.at[pl.ds(off, CH)], didx)

    load_idx(base, sidx0, didx0)
    pltpu.async_copy(y_hbm.at[sidx0], rows0, sem0)

    def pair(j, _):
        i = j * 2
        load_idx(base + (i + 1) * CH, sidx1, didx1)
        pltpu.async_copy(y_hbm.at[sidx1], rows1, sem1)
        pltpu.make_async_copy(y_hbm.at[sidx0], rows0, sem0).wait()
        pltpu.sync_copy(rows0, zsh.at[didx0], add=True)
        load_idx(base + (i + 2) * CH, sidx0, didx0)
        pltpu.async_copy(y_hbm.at[sidx0], rows0, sem0)
        pltpu.make_async_copy(y_hbm.at[sidx1], rows1, sem1).wait()
        pltpu.sync_copy(rows1, zsh.at[didx1], add=True)
        return 0

    lax.fori_loop(0, (NCH - 1) // 2, pair, 0)
    pltpu.make_async_copy(y_hbm.at[sidx0], rows0, sem0).wait()
    pltpu.sync_copy(rows0, zsh.at[didx0], add=True)
    plsc.subcore_barrier()
    _staged_readout(zsh, rows0, z_hbm, c, s)


@functools.partial(
    pl.kernel,
    mesh=_SC_MESH,
    out_type=jax.ShapeDtypeStruct((2, N, CW), jnp.float32),
    scratch_types=[
        pltpu.VMEM((CH,), jnp.int32),
        pltpu.VMEM((CH, CW), jnp.float32),
        pltpu.VMEM_SHARED((ZR, CW), jnp.float32),
    ],
)
def _sc_cnt(idx_hbm, c_hbm, cidx, obuf, csh):
    c = lax.axis_index("c")
    s = lax.axis_index("s")
    wid = s * 2 + c

    _zero_rows(obuf, CH, CW)
    for t in range(RPT // CH):
        pltpu.sync_copy(obuf, csh.at[pl.ds(s * RPT + t * CH, CH)])
    plsc.subcore_barrier()

    one = jnp.ones((16,), jnp.float32)

    def fill(i, _):
        for j in range(CW // 16):
            obuf[i, pl.ds(j * 16, 16)] = one
        return 0

    lax.fori_loop(0, CH, fill, 0)

    def body(i, _):
        off = wid * EPW + i * CH
        pltpu.sync_copy(idx_hbm.at[pl.ds(off, CH)], cidx)
        pltpu.sync_copy(obuf, csh.at[cidx], add=True)
        return 0

    lax.fori_loop(0, NCH, body, 0)
    plsc.subcore_barrier()
    _staged_readout(csh, obuf, c_hbm, c, s)


def _relu(v):
    return jnp.maximum(v, 0.0)


def _mm(a, b):
    return jnp.dot(a, b, preferred_element_type=jnp.float32)


def _tc_in_body(x, w1, b1, w2, b2, wg, cp, wr, oh, oy, od):
    h = _mm(_relu(_mm(x[...], w1[...]) + b1[...]), w2[...]) + b2[...]
    cnt = (cp[0] + cp[1])[:, 0:1]
    dinv = lax.rsqrt(wr[0, 0] * cnt + 2.0)
    oh[...] = h
    od[...] = dinv
    oy[...] = dinv * _mm(h, wg[...])


def _tc_in(x, w1, b1, w2, b2, wg, cp, wr):
    return pl.pallas_call(
        _tc_in_body,
        out_shape=(
            jax.ShapeDtypeStruct((N, H), jnp.float32),
            jax.ShapeDtypeStruct((N, H), jnp.float32),
            jax.ShapeDtypeStruct((N, 1), jnp.float32),
        ),
    )(x, w1, b1.reshape(1, H), w2, b2.reshape(1, H), wg, cp, wr)


def _tc_post_body(has_skip, is_final, *refs):
    if is_final:
        (zp, y, h, dinv, wr, gb, ms, nw, nb, skw, skb,
         hw1, hb1, hw2, hb2, hw3, hb3, out) = refs
    elif has_skip:
        (zp, y, h, dinv, wr, gb, ms, nw, nb, skw, skb, wgn, oh, oy) = refs
    else:
        (zp, y, h, dinv, wr, gb, ms, nw, nb, wgn, oh, oy) = refs
    di = dinv[...]
    g = di * (wr[0, 0] * (zp[0] + zp[1]) + 2.0 * y[...]) + gb[...]
    mean = jnp.mean(g, axis=0, keepdims=True)
    o = g - ms[...] * mean
    var = jnp.mean(o * o, axis=0, keepdims=True)
    g = nw[...] * o * lax.rsqrt(var + 1e-5) + nb[...]
    g = jnp.where(g > 0, g, jnp.exp(g) - 1.0)
    if has_skip:
        g = g + _mm(h[...], skw[...]) + skb[...]
    if is_final:
        q = _relu(_mm(g, hw1[...]) + hb1[...])
        q = _relu(_mm(q, hw2[...]) + hb2[...])
        q = _mm(q, hw3[...]) + hb3[...]
        out[...] = 1.0 / (1.0 + jnp.exp(-q))
    else:
        oh[...] = g
        oy[...] = di * _mm(g, wgn[...])


def _tc_mid(has_skip, args):
    return pl.pallas_call(
        functools.partial(_tc_post_body, has_skip, False),
        out_shape=(
            jax.ShapeDtypeStruct((N, H), jnp.float32),
            jax.ShapeDtypeStruct((N, H), jnp.float32),
        ),
    )(*args)


def _tc_final(args):
    return pl.pallas_call(
        functools.partial(_tc_post_body, True, True),
        out_shape=jax.ShapeDtypeStruct((N, 1), jnp.float32),
    )(*args)


def kernel(x, edge_index, batch, params):
    p = params
    wr = p['edge_weight'].reshape(1, 1)
    row, col = edge_index[0], edge_index[1]
    pad0 = jnp.zeros((PAD,), jnp.int32)
    padd = jnp.full((PAD,), DUMMY, jnp.int32)
    srcg = jnp.concatenate([row, col, pad0])
    cidx = jnp.concatenate([row, col, padd])
    dst = jnp.concatenate([col, row, padd])

    cp = _sc_cnt(cidx)
    h, y, dinv = _tc_in(x, p['in_W1'], p['in_b1'], p['in_W2'], p['in_b2'],
                        p['gcn_W0'], cp, wr)

    for i in range(NUM_LAYERS):
        zp = _sc_msg(y, srcg, dst)
        gb = p['gcn_b%d' % i].reshape(1, H)
        ms = p['norm_ms%d' % i].reshape(1, H)
        nw = p['norm_w%d' % i].reshape(1, H)
        nb = p['norm_b%d' % i].reshape(1, H)
        if i < NUM_LAYERS - 1:
            args = [zp, y, h, dinv, wr, gb, ms, nw, nb]
            has_skip = i % RES_FREQ == 0
            if has_skip:
                args += [p['skip_W%d' % i], p['skip_b%d' % i].reshape(1, H)]
            args += [p['gcn_W%d' % (i + 1)]]
            h, y = _tc_mid(has_skip, args)
        else:
            args = [zp, y, h, dinv, wr, gb, ms, nw, nb,
                    p['skip_W%d' % i], p['skip_b%d' % i].reshape(1, H),
                    p['head_W1'], p['head_b1'].reshape(1, H),
                    p['head_W2'], p['head_b2'].reshape(1, H // 2),
                    p['head_W3'], p['head_b3'].reshape(1, 1)]
            out = _tc_final(args)
    return out.reshape(-1)
